# baseline (speedup 1.0000x reference)
import functools

import jax
import jax.numpy as jnp
from jax import lax
from jax.experimental import pallas as pl
from jax.experimental.pallas import tpu as pltpu
from jax.experimental.pallas import tpu_sc as plsc

_NC = 2
_NS = 16
_NW = _NC * _NS
_L = 16

_T = 1000
_F = 256
_U = 4
_CAP = 1280


def _build_sc_call(Bn, N, Dd, E):
    ROWS = (N + _NW - 1) // _NW
    NPAD = ROWS * _NW
    assert E % (2 * _T) == 0
    assert Dd % _L == 0
    NBLK = E // _T
    NDJ = Dd // _L

    mesh = plsc.VectorSubcoreMesh(core_axis_name="c", subcore_axis_name="s")

    @functools.partial(
        pl.kernel,
        out_type=jax.ShapeDtypeStruct((Bn * NPAD * Dd,), jnp.float32),
        mesh=mesh,
        compiler_params=pltpu.CompilerParams(needs_layout_passes=False),
        scratch_types=[
            pltpu.VMEM((_T,), jnp.int32),
            pltpu.VMEM((_T,), jnp.int32),
            pltpu.VMEM((_T,), jnp.int32),
            pltpu.VMEM((_T,), jnp.int32),
            pltpu.VMEM((_T,), jnp.float32),
            pltpu.VMEM((_T,), jnp.float32),
            pltpu.VMEM((_T,), jnp.float32),
            pltpu.VMEM((_T,), jnp.float32),
            pltpu.VMEM((_CAP,), jnp.int32),
            pltpu.VMEM((_CAP,), jnp.int32),
            pltpu.VMEM((_CAP,), jnp.float32),
            pltpu.VMEM((_CAP,), jnp.float32),
            pltpu.VMEM((_F, Dd), jnp.float32),
            pltpu.VMEM((ROWS * Dd,), jnp.float32),
            pltpu.VMEM((ROWS * Dd,), jnp.float32),
            pltpu.SemaphoreType.DMA,
            pltpu.SemaphoreType.DMA,
        ],
    )
    def call(x_hbm, dst_hbm, src_hbm, a00_hbm, a01_hbm, a10_hbm, a11_hbm,
             out_hbm, dst_va, dst_vb, src_va, src_vb, a0_va, a0_vb,
             a1_va, a1_vb, msrc_v, mdst_v, ma0_v, ma1_v, rows_v, acc0, acc1,
             sem_a, sem_b):
        dst_v = (dst_va, dst_vb)
        src_v = (src_va, src_vb)
        a0_v = (a0_va, a0_vb)
        a1_v = (a1_va, a1_vb)
        w = lax.axis_index("s") * _NC + lax.axis_index("c")
        lo = w * ROWS
        urows = jnp.uint32(ROWS)

        @pl.loop(0, _CAP, step=_L)
        def _(i):
            msrc_v[pl.ds(i, _L)] = jnp.zeros((_L,), jnp.int32)

        def process_pending(cnt):
            mdst_v[pl.ds(cnt, _L)] = jnp.full((_L,), lo, jnp.int32)
            ma0_v[pl.ds(cnt, _L)] = jnp.zeros((_L,), jnp.float32)
            ma1_v[pl.ds(cnt, _L)] = jnp.zeros((_L,), jnp.float32)
            nchunks = (cnt + _F - 1) // _F

            def chunk_fn(ci, _):
                c0 = ci * _F
                pltpu.sync_copy(x_hbm.at[msrc_v.at[pl.ds(c0, _F)]], rows_v)
                m = jnp.minimum(_F, cnt - c0)
                mu = (m + _U - 1) // _U

                def edge_fn(q, __):
                    kk = q * _U
                    k = c0 + kk
                    for u in range(_U):
                        roff = (mdst_v[pl.ds(k + u, _L)][0] - lo) * Dd
                        s0 = ma0_v[pl.ds(k + u, _L)][0]
                        s1 = ma1_v[pl.ds(k + u, _L)][0]
                        for j in range(NDJ):
                            xv = rows_v[kk + u, pl.ds(j * _L, _L)]
                            sl = pl.ds(roff + j * _L, _L)
                            acc0[sl] = jnp.maximum(acc0[sl], s0 * xv)
                            acc1[sl] = jnp.maximum(acc1[sl], s1 * xv)
                    return 0

                lax.fori_loop(0, mu, edge_fn, 0)
                return 0

            lax.fori_loop(0, nchunks, chunk_fn, 0)

        def run_batch(b, a0s_hbm, a1s_hbm, src_off):
            @pl.loop(0, ROWS * Dd, step=_L)
            def _(i):
                acc0[pl.ds(i, _L)] = jnp.zeros((_L,), jnp.float32)
                acc1[pl.ds(i, _L)] = jnp.zeros((_L,), jnp.float32)

            def start_block(blk, p, sem):
                base = blk * _T
                pltpu.async_copy(dst_hbm.at[pl.ds(base, _T)], dst_v[p], sem)
                pltpu.async_copy(src_hbm.at[pl.ds(base, _T)], src_v[p], sem)
                pltpu.async_copy(a0s_hbm.at[pl.ds(base, _T)], a0_v[p], sem)
                pltpu.async_copy(a1s_hbm.at[pl.ds(base, _T)], a1_v[p], sem)

            def wait_block(p, sem):
                pltpu.make_async_copy(dst_hbm.at[pl.ds(0, _T)], dst_v[p], sem).wait()
                pltpu.make_async_copy(src_hbm.at[pl.ds(0, _T)], src_v[p], sem).wait()
                pltpu.make_async_copy(a0s_hbm.at[pl.ds(0, _T)], a0_v[p], sem).wait()
                pltpu.make_async_copy(a1s_hbm.at[pl.ds(0, _T)], a1_v[p], sem).wait()

            def scan_block(p, cnt):
                def vec_fn(i, cnt):
                    o = i * _L
                    dv = dst_v[p][pl.ds(o, _L)]
                    mask = (dv - lo).astype(jnp.uint32) < urows
                    nm = plsc.all_reduce_population_count(mask)[0]

                    @pl.when(nm > 0)
                    def _():
                        plsc.store_compressed(mdst_v.at[pl.ds(cnt, _L)], dv,
                                              mask=mask)
                        sv = src_v[p][pl.ds(o, _L)] + src_off
                        plsc.store_compressed(msrc_v.at[pl.ds(cnt, _L)], sv,
                                              mask=mask)
                        plsc.store_compressed(ma0_v.at[pl.ds(cnt, _L)],
                                              a0_v[p][pl.ds(o, _L)], mask=mask)
                        plsc.store_compressed(ma1_v.at[pl.ds(cnt, _L)],
                                              a1_v[p][pl.ds(o, _L)], mask=mask)

                    return cnt + nm

                return lax.fori_loop(0, _T // _L, vec_fn, cnt)

            def maybe_flush(cnt):
                @pl.when(cnt >= _F)
                def _():
                    process_pending(cnt)

                return jnp.where(cnt >= _F, 0, cnt)

            start_block(0, 0, sem_a)
            start_block(1, 1, sem_b)

            def pair_fn(g, cnt):
                blk = g * 2
                wait_block(0, sem_a)
                cnt = scan_block(0, cnt)

                @pl.when(blk + 2 < NBLK)
                def _():
                    start_block(blk + 2, 0, sem_a)

                cnt = maybe_flush(cnt)
                wait_block(1, sem_b)
                cnt = scan_block(1, cnt)

                @pl.when(blk + 3 < NBLK)
                def _():
                    start_block(blk + 3, 1, sem_b)

                return maybe_flush(cnt)

            cnt = lax.fori_loop(0, NBLK // 2, pair_fn, 0)
            process_pending(cnt)

            @pl.loop(0, ROWS * Dd, step=_L)
            def _(i):
                acc0[pl.ds(i, _L)] = acc0[pl.ds(i, _L)] + acc1[pl.ds(i, _L)]

            out_base = b * NPAD * Dd + lo * Dd
            pltpu.sync_copy(acc0, out_hbm.at[pl.ds(out_base, ROWS * Dd)])

        run_batch(0, a00_hbm, a01_hbm, 0)
        run_batch(1, a10_hbm, a11_hbm, N)

    return call, ROWS, NPAD


def kernel(inputs, edge_index, adj_values):
    Bn, N, Dd = inputs.shape
    E = adj_values.shape[2]
    ei = edge_index.astype(jnp.int32)
    x2 = inputs.reshape(Bn * N, Dd)
    call, _, NPAD = _build_sc_call(Bn, N, Dd, E)
    out_flat = call(x2, ei[0], ei[1],
                    adj_values[0, 0], adj_values[0, 1],
                    adj_values[1, 0], adj_values[1, 1])
    return out_flat.reshape(Bn, NPAD, Dd)[:, :N, :]


# speedup vs baseline: 2.2258x; 2.2258x over previous
import functools

import jax
import jax.numpy as jnp
from jax import lax
from jax.experimental import pallas as pl
from jax.experimental.pallas import tpu as pltpu
from jax.experimental.pallas import tpu_sc as plsc

_NC = 2
_NS = 16
_NW = _NC * _NS
_L = 16

_T = 1000
_F = 128
_FLUSH = 256
_U = 4
_CAP = 1280


def _build_sc_call(Bn, N, Dd, E):
    ROWS = (N + _NW - 1) // _NW
    NPAD = ROWS * _NW
    assert E % (2 * _T) == 0
    assert Dd % _L == 0
    NBLK = E // _T
    NDJ = Dd // _L

    mesh = plsc.VectorSubcoreMesh(core_axis_name="c", subcore_axis_name="s")

    @functools.partial(
        pl.kernel,
        out_type=jax.ShapeDtypeStruct((Bn * NPAD * Dd,), jnp.float32),
        mesh=mesh,
        compiler_params=pltpu.CompilerParams(needs_layout_passes=False),
        scratch_types=[
            pltpu.VMEM((_T,), jnp.int32),
            pltpu.VMEM((_T,), jnp.int32),
            pltpu.VMEM((_T,), jnp.int32),
            pltpu.VMEM((_T,), jnp.int32),
            pltpu.VMEM((_T,), jnp.float32),
            pltpu.VMEM((_T,), jnp.float32),
            pltpu.VMEM((_T,), jnp.float32),
            pltpu.VMEM((_T,), jnp.float32),
            pltpu.VMEM((_CAP,), jnp.int32),
            pltpu.VMEM((_CAP,), jnp.int32),
            pltpu.VMEM((_CAP,), jnp.float32),
            pltpu.VMEM((_CAP,), jnp.float32),
            pltpu.VMEM((2 * _F, Dd), jnp.float32),
            pltpu.VMEM((ROWS * Dd,), jnp.float32),
            pltpu.VMEM((ROWS * Dd,), jnp.float32),
            pltpu.SemaphoreType.DMA,
            pltpu.SemaphoreType.DMA,
            pltpu.SemaphoreType.DMA,
        ],
    )
    def call(x_hbm, dst_hbm, src_hbm, a00_hbm, a01_hbm, a10_hbm, a11_hbm,
             out_hbm, dst_va, dst_vb, src_va, src_vb, a0_va, a0_vb,
             a1_va, a1_vb, msrc_v, mdst_v, ma0_v, ma1_v, rows_v, acc0, acc1,
             sem_a, sem_b, gsem):
        dst_v = (dst_va, dst_vb)
        src_v = (src_va, src_vb)
        a0_v = (a0_va, a0_vb)
        a1_v = (a1_va, a1_vb)
        w = lax.axis_index("s") * _NC + lax.axis_index("c")
        lo = w * ROWS
        urows = jnp.uint32(ROWS)

        @pl.loop(0, _CAP, step=_L)
        def _(i):
            msrc_v[pl.ds(i, _L)] = jnp.zeros((_L,), jnp.int32)

        def process_pending(cnt):
            mdst_v[pl.ds(cnt, _L)] = jnp.full((_L,), lo, jnp.int32)
            ma0_v[pl.ds(cnt, _L)] = jnp.zeros((_L,), jnp.float32)
            ma1_v[pl.ds(cnt, _L)] = jnp.zeros((_L,), jnp.float32)
            nchunks = (cnt + _F - 1) // _F

            def start_g(ci, slot):
                pltpu.async_copy(
                    x_hbm.at[msrc_v.at[pl.ds(ci * _F, _F)]],
                    rows_v.at[pl.ds(slot * _F, _F)], gsem)

            def wait_g(slot):
                pltpu.make_async_copy(
                    x_hbm.at[msrc_v.at[pl.ds(0, _F)]],
                    rows_v.at[pl.ds(slot * _F, _F)], gsem).wait()

            def body(ci, slot):
                c0 = ci * _F
                m = jnp.minimum(_F, cnt - c0)
                mu = (m + _U - 1) // _U
                base = slot * _F

                def edge_fn(q, __):
                    kk = q * _U
                    k = c0 + kk
                    for u in range(_U):
                        roff = (mdst_v[pl.ds(k + u, _L)][0] - lo) * Dd
                        s0 = ma0_v[pl.ds(k + u, _L)][0]
                        s1 = ma1_v[pl.ds(k + u, _L)][0]
                        for j in range(NDJ):
                            xv = rows_v[base + kk + u, pl.ds(j * _L, _L)]
                            sl = pl.ds(roff + j * _L, _L)
                            acc0[sl] = jnp.maximum(acc0[sl], s0 * xv)
                            acc1[sl] = jnp.maximum(acc1[sl], s1 * xv)
                    return 0

                lax.fori_loop(0, mu, edge_fn, 0)

            @pl.when(nchunks > 0)
            def _():
                start_g(0, 0)

            def chunk_fn(ci, _):
                @pl.when(lax.rem(ci, 2) == 0)
                def _():
                    wait_g(0)

                    @pl.when(ci + 1 < nchunks)
                    def _():
                        start_g(ci + 1, 1)

                    body(ci, 0)

                @pl.when(lax.rem(ci, 2) == 1)
                def _():
                    wait_g(1)

                    @pl.when(ci + 1 < nchunks)
                    def _():
                        start_g(ci + 1, 0)

                    body(ci, 1)

                return 0

            lax.fori_loop(0, nchunks, chunk_fn, 0)

        def run_batch(b, a0s_hbm, a1s_hbm, src_off):
            @pl.loop(0, ROWS * Dd, step=_L)
            def _(i):
                acc0[pl.ds(i, _L)] = jnp.zeros((_L,), jnp.float32)
                acc1[pl.ds(i, _L)] = jnp.zeros((_L,), jnp.float32)

            def start_block(blk, p, sem):
                base = blk * _T
                pltpu.async_copy(dst_hbm.at[pl.ds(base, _T)], dst_v[p], sem)
                pltpu.async_copy(src_hbm.at[pl.ds(base, _T)], src_v[p], sem)
                pltpu.async_copy(a0s_hbm.at[pl.ds(base, _T)], a0_v[p], sem)
                pltpu.async_copy(a1s_hbm.at[pl.ds(base, _T)], a1_v[p], sem)

            def wait_block(p, sem):
                pltpu.make_async_copy(dst_hbm.at[pl.ds(0, _T)], dst_v[p], sem).wait()
                pltpu.make_async_copy(src_hbm.at[pl.ds(0, _T)], src_v[p], sem).wait()
                pltpu.make_async_copy(a0s_hbm.at[pl.ds(0, _T)], a0_v[p], sem).wait()
                pltpu.make_async_copy(a1s_hbm.at[pl.ds(0, _T)], a1_v[p], sem).wait()

            def scan_block(p, cnt):
                def vec_fn(i, cnt):
                    o = i * _L
                    dv = dst_v[p][pl.ds(o, _L)]
                    mask = (dv - lo).astype(jnp.uint32) < urows
                    nm = plsc.all_reduce_population_count(mask)[0]

                    @pl.when(nm > 0)
                    def _():
                        plsc.store_compressed(mdst_v.at[pl.ds(cnt, _L)], dv,
                                              mask=mask)
                        sv = src_v[p][pl.ds(o, _L)] + src_off
                        plsc.store_compressed(msrc_v.at[pl.ds(cnt, _L)], sv,
                                              mask=mask)
                        plsc.store_compressed(ma0_v.at[pl.ds(cnt, _L)],
                                              a0_v[p][pl.ds(o, _L)], mask=mask)
                        plsc.store_compressed(ma1_v.at[pl.ds(cnt, _L)],
                                              a1_v[p][pl.ds(o, _L)], mask=mask)

                    return cnt + nm

                return lax.fori_loop(0, _T // _L, vec_fn, cnt)

            def maybe_flush(cnt):
                @pl.when(cnt >= _FLUSH)
                def _():
                    process_pending(cnt)

                return jnp.where(cnt >= _FLUSH, 0, cnt)

            start_block(0, 0, sem_a)
            start_block(1, 1, sem_b)

            def pair_fn(g, cnt):
                blk = g * 2
                wait_block(0, sem_a)
                cnt = scan_block(0, cnt)

                @pl.when(blk + 2 < NBLK)
                def _():
                    start_block(blk + 2, 0, sem_a)

                cnt = maybe_flush(cnt)
                wait_block(1, sem_b)
                cnt = scan_block(1, cnt)

                @pl.when(blk + 3 < NBLK)
                def _():
                    start_block(blk + 3, 1, sem_b)

                return maybe_flush(cnt)

            cnt = lax.fori_loop(0, NBLK // 2, pair_fn, 0)
            process_pending(cnt)

            @pl.loop(0, ROWS * Dd, step=_L)
            def _(i):
                acc0[pl.ds(i, _L)] = acc0[pl.ds(i, _L)] + acc1[pl.ds(i, _L)]

            out_base = b * NPAD * Dd + lo * Dd
            pltpu.sync_copy(acc0, out_hbm.at[pl.ds(out_base, ROWS * Dd)])

        run_batch(0, a00_hbm, a01_hbm, 0)
        run_batch(1, a10_hbm, a11_hbm, N)

    return call, ROWS, NPAD


def kernel(inputs, edge_index, adj_values):
    Bn, N, Dd = inputs.shape
    E = adj_values.shape[2]
    ei = edge_index.astype(jnp.int32)
    x2 = inputs.reshape(Bn * N, Dd)
    call, _, NPAD = _build_sc_call(Bn, N, Dd, E)
    out_flat = call(x2, ei[0], ei[1],
                    adj_values[0, 0], adj_values[0, 1],
                    adj_values[1, 0], adj_values[1, 1])
    return out_flat.reshape(Bn, NPAD, Dd)[:, :N, :]
